# Initial kernel scaffold; baseline (speedup 1.0000x reference)
import jax
import jax.numpy as jnp
from jax.experimental import pallas as pl


def kernel(data, segments, num_segments, ctx, W, b):
    raise NotImplementedError("write your pallas kernel here")


# speedup vs baseline: 20.6311x; 20.6311x over previous
import functools

import jax
import jax.numpy as jnp
import numpy as np
from jax.experimental import pallas as pl
from jax.experimental.pallas import tpu as pltpu

N = 160000
D = 128
INNER = 16
NSEG = 10000
NET_OUT = INNER * (INNER + 1) // 2 + INNER
NTRI = NET_OUT - INNER
FF = INNER * INNER

B = 512
NB = 313
NBP = 320
NPAD = NB * B

SB = 1000
NSB = NSEG // SB

_tri_i, _tri_j = np.tril_indices(INNER)
_P = np.zeros((NTRI, FF), np.float32)
for _idx, (_i, _j) in enumerate(zip(_tri_i, _tri_j)):
    _P[_idx, INNER * _i + _j] = 1.0
_DMASK = np.zeros((1, FF), np.float32)
for _i in range(INNER):
    _DMASK[0, INNER * _i + _i] = 1.0

_SALL = np.zeros((FF, FF), np.float32)
_RESP = np.zeros((INNER, FF), np.float32)
_TILE = np.zeros((INNER, FF), np.float32)
for _i in range(INNER):
    for _k in range(INNER):
        _SALL[INNER * _i + _k, INNER * _k + _i] = 1.0
    for _j in range(INNER):
        _RESP[_i, INNER * _i + _j] = 1.0
        _TILE[_j, INNER * _i + _j] = 1.0


def _expand_reduce_kernel(nu_ref, tab_ref, dat_ref, lr_ref, wt_ref, wy_ref,
                          bt_ref, by_ref, p_ref, dm_ref, outF_ref, outt_ref,
                          cF_ref, ct_ref):
    j = pl.program_id(0)

    @pl.when(j == 0)
    def _zero():
        outF_ref[...] = jnp.zeros_like(outF_ref)
        outt_ref[...] = jnp.zeros_like(outt_ref)

    dat = dat_ref[...]
    t = dat @ wt_ref[...] + bt_ref[...]
    y = jnp.tanh(dat @ wy_ref[...] + by_ref[...])
    L = y @ p_ref[...]
    L = jnp.where(dm_ref[...] > 0.0, jax.nn.softplus(L), L)
    Lm = L.reshape(B, INNER, INNER)
    F3 = jax.lax.dot_general(Lm, Lm, (((2,), (2,)), ((0,), (0,))),
                             preferred_element_type=jnp.float32)
    F = F3.reshape(B, FF)

    rowid = j * B + jax.lax.broadcasted_iota(jnp.int32, (B, 1), 0)
    valid = (rowid < N).astype(jnp.float32)
    F = F * valid
    t = t * valid

    lr = lr_ref[...]
    oh = (lr == jax.lax.broadcasted_iota(jnp.int32, (B, B), 1)
          ).astype(jnp.float32)
    cF_ref[...] = jax.lax.dot_general(oh, F, (((0,), (0,)), ((), ())),
                                      preferred_element_type=jnp.float32)
    ct_ref[...] = jax.lax.dot_general(oh, t, (((0,), (0,)), ((), ())),
                                      preferred_element_type=jnp.float32)

    jj = j % 8

    def body(u, carry):
        sid = tab_ref[jj, u]
        fF = cF_ref[pl.ds(u, 1), :]
        ft = ct_ref[pl.ds(u, 1), :]
        outF_ref[pl.ds(sid, 1), :] = outF_ref[pl.ds(sid, 1), :] + fF
        outt_ref[pl.ds(sid, 1), :] = outt_ref[pl.ds(sid, 1), :] + ft
        return carry

    jax.lax.fori_loop(0, nu_ref[jj, 0], body, 0)


def _solve_kernel(F_ref, t_ref, dm_ref, s_ref, r_ref, tl_ref, o_ref):
    A = F_ref[...] + dm_ref[...]
    bv = jnp.ones((SB, INNER), jnp.float32)
    colid = jax.lax.broadcasted_iota(jnp.int32, (1, FF), 1)
    colid16 = jax.lax.broadcasted_iota(jnp.int32, (1, INNER), 1)
    rmat = r_ref[...]
    tmat = tl_ref[...]
    for k in range(INNER):
        arow = A[:, INNER * k:INNER * (k + 1)]
        p = arow[:, k:k + 1]
        arow = arow / p
        brow = bv[:, k:k + 1] / p
        sk = s_ref[:, INNER * k:INNER * (k + 1)]
        f = jnp.dot(A, sk, preferred_element_type=jnp.float32)
        f_exp = jnp.dot(f, rmat, preferred_element_type=jnp.float32)
        arow_t = jnp.dot(arow, tmat, preferred_element_type=jnp.float32)
        Anew = A - f_exp * arow_t
        rowmask = (colid // INNER) == k
        A = jnp.where(rowmask, arow_t, Anew)
        bv = jnp.where(colid16 == k, brow, bv - f * brow)
    o_ref[...] = bv * t_ref[...]


def kernel(data, segments, num_segments, ctx, W, b):
    del num_segments, ctx
    seg = segments.astype(jnp.int32)
    pad = NPAD - N
    segp = jnp.concatenate([seg, jnp.full((pad,), seg[-1], jnp.int32)])
    datap = jnp.concatenate(
        [data, jnp.zeros((pad, D), jnp.float32)], axis=0)

    boundary = jnp.concatenate(
        [jnp.ones((1,), jnp.bool_), segp[1:] != segp[:-1]])
    grank = jnp.cumsum(boundary.astype(jnp.int32)) - 1
    r0 = grank[::B]
    lrank = (grank - jnp.repeat(r0, B)).reshape(NPAD, 1)
    nuniq = (grank[B - 1::B] - r0 + 1).reshape(NB, 1)
    nuniq = jnp.concatenate(
        [nuniq, jnp.zeros((NBP - NB, 1), jnp.int32)], axis=0)
    uniq_seg = jnp.zeros((NSEG,), jnp.int32).at[grank].set(segp)
    tab = uniq_seg[jnp.clip(r0[:, None] + jnp.arange(B)[None, :],
                            0, NSEG - 1)]
    tab = jnp.concatenate(
        [tab, jnp.zeros((NBP - NB, B), jnp.int32)], axis=0)

    Wt = W[:, :INNER]
    Wy = W[:, INNER:]
    bt = b[:INNER].reshape(1, INNER)
    by = b[INNER:].reshape(1, NTRI)
    P = jnp.asarray(_P)
    dmask = jnp.asarray(_DMASK)

    outF, outt = pl.pallas_call(
        _expand_reduce_kernel,
        grid=(NB,),
        in_specs=[
            pl.BlockSpec((8, 1), lambda j: (j // 8, 0),
                         memory_space=pltpu.SMEM),
            pl.BlockSpec((8, B), lambda j: (j // 8, 0),
                         memory_space=pltpu.SMEM),
            pl.BlockSpec((B, D), lambda j: (j, 0)),
            pl.BlockSpec((B, 1), lambda j: (j, 0)),
            pl.BlockSpec((D, INNER), lambda j: (0, 0)),
            pl.BlockSpec((D, NTRI), lambda j: (0, 0)),
            pl.BlockSpec((1, INNER), lambda j: (0, 0)),
            pl.BlockSpec((1, NTRI), lambda j: (0, 0)),
            pl.BlockSpec((NTRI, FF), lambda j: (0, 0)),
            pl.BlockSpec((1, FF), lambda j: (0, 0)),
        ],
        out_specs=[
            pl.BlockSpec((NSEG, FF), lambda j: (0, 0)),
            pl.BlockSpec((NSEG, INNER), lambda j: (0, 0)),
        ],
        out_shape=[
            jax.ShapeDtypeStruct((NSEG, FF), jnp.float32),
            jax.ShapeDtypeStruct((NSEG, INNER), jnp.float32),
        ],
        scratch_shapes=[
            pltpu.VMEM((B, FF), jnp.float32),
            pltpu.VMEM((B, INNER), jnp.float32),
        ],
    )(nuniq, tab, datap, lrank, Wt, Wy, bt, by, P, dmask)

    theta = pl.pallas_call(
        _solve_kernel,
        grid=(NSB,),
        in_specs=[
            pl.BlockSpec((SB, FF), lambda j: (j, 0)),
            pl.BlockSpec((SB, INNER), lambda j: (j, 0)),
            pl.BlockSpec((1, FF), lambda j: (0, 0)),
            pl.BlockSpec((FF, FF), lambda j: (0, 0)),
            pl.BlockSpec((INNER, FF), lambda j: (0, 0)),
            pl.BlockSpec((INNER, FF), lambda j: (0, 0)),
        ],
        out_specs=pl.BlockSpec((SB, INNER), lambda j: (j, 0)),
        out_shape=jax.ShapeDtypeStruct((NSEG, INNER), jnp.float32),
    )(outF, outt, dmask, jnp.asarray(_SALL), jnp.asarray(_RESP),
      jnp.asarray(_TILE))
    return theta
